# baseline (speedup 1.0000x reference)
import jax
import jax.numpy as jnp
from jax.experimental import pallas as pl
from jax.experimental.pallas import tpu as pltpu

N = 10000
F_IN = 128
F_HID = 64
BM = 200
NS = 2


def _gcn_kernel(x_ref, w_ref, b_ref, a0, a1, logp_ref, embed_ref, support_ref):
    @pl.when(pl.program_id(0) == 0)
    def _():
        support_ref[:] = jnp.dot(
            x_ref[:], w_ref[:], preferred_element_type=jnp.float32
        )

    for s, a in enumerate((a0, a1)):
        out = jnp.dot(a[:], support_ref[:], preferred_element_type=jnp.float32)
        out = out + b_ref[:]
        embed_ref[s * BM:(s + 1) * BM, :] = out
        m = jnp.max(out, axis=1, keepdims=True)
        lse = jnp.log(jnp.sum(jnp.exp(out - m), axis=1, keepdims=True)) + m
        logp_ref[s * BM:(s + 1) * BM, :] = out - lse


def kernel(x, adj, W, b):
    b2 = b.reshape(1, F_HID)
    logp, embed = pl.pallas_call(
        _gcn_kernel,
        grid=(N // (BM * NS),),
        in_specs=[
            pl.BlockSpec((N, F_IN), lambda i: (0, 0)),
            pl.BlockSpec((F_IN, F_HID), lambda i: (0, 0)),
            pl.BlockSpec((1, F_HID), lambda i: (0, 0)),
            pl.BlockSpec((BM, N), lambda i: (2 * i, 0)),
            pl.BlockSpec((BM, N), lambda i: (2 * i + 1, 0)),
        ],
        out_specs=[
            pl.BlockSpec((BM * NS, F_HID), lambda i: (i, 0)),
            pl.BlockSpec((BM * NS, F_HID), lambda i: (i, 0)),
        ],
        out_shape=[
            jax.ShapeDtypeStruct((N, F_HID), jnp.float32),
            jax.ShapeDtypeStruct((N, F_HID), jnp.float32),
        ],
        scratch_shapes=[pltpu.VMEM((N, F_HID), jnp.float32)],
        compiler_params=pltpu.CompilerParams(
            dimension_semantics=("arbitrary",),
        ),
    )(x, W, b2, adj, adj)
    return (logp, embed)
